# baseline (speedup 1.0000x reference)
import jax
import jax.numpy as jnp
from jax import lax
from jax.experimental import pallas as pl
from jax.experimental.pallas import tpu as pltpu

_NCODES = 1024
_HW = 1024
_D = 64
_NIMG = 16
_NELEM = float(_NIMG * _HW * _D)


def _vq_body(z_ref, emb_ref, embm2_ref, embT_ref, idx_ref, zq_ref, loss_ref,
             en_ref):
    n = pl.program_id(0)
    z2d = z_ref[0]
    zn = jnp.sum(z2d * z2d, axis=0, keepdims=True)

    @pl.when(n == 0)
    def _():
        emb = emb_ref[...]
        en_ref[...] = jnp.sum(emb * emb, axis=1, keepdims=True)

    en = en_ref[...]
    dotm2 = lax.dot_general(
        embm2_ref[...], z2d, (((1,), (0,)), ((), ())),
        preferred_element_type=jnp.float32,
    )
    dist = (zn + en) + dotm2
    m = jnp.min(dist, axis=0, keepdims=True)
    kio = lax.broadcasted_iota(jnp.int32, (_NCODES, _HW), 0)
    idx = jnp.min(jnp.where(dist == m, kio, _NCODES), axis=0, keepdims=True)
    idx_ref[0] = idx
    oh = jnp.where(kio == idx, 1.0, 0.0)
    zqT = lax.dot_general(
        embT_ref[...], oh, (((1,), (0,)), ((), ())),
        preferred_element_type=jnp.float32,
        precision=lax.Precision.HIGHEST,
    )
    zq_ref[0] = zqT
    part = jnp.sum(m, keepdims=True).reshape(1, 1)
    prev = jnp.where(n == 0, jnp.zeros((1, 1), jnp.float32), loss_ref[...])
    total = prev + part
    loss_ref[...] = jnp.where(n == _NIMG - 1, total / _NELEM, total)


def kernel(z_e, embedding):
    z_r = z_e.reshape(_NIMG, _D, _HW)
    embm2 = embedding * -2.0
    embT = embedding.T
    idx3, zq, loss = pl.pallas_call(
        _vq_body,
        grid=(_NIMG,),
        in_specs=[
            pl.BlockSpec((1, _D, _HW), lambda n: (n, 0, 0)),
            pl.BlockSpec((_NCODES, _D), lambda n: (0, 0)),
            pl.BlockSpec((_NCODES, _D), lambda n: (0, 0)),
            pl.BlockSpec((_D, _NCODES), lambda n: (0, 0)),
        ],
        out_specs=(
            pl.BlockSpec((1, 1, _HW), lambda n: (n, 0, 0)),
            pl.BlockSpec((1, _D, _HW), lambda n: (n, 0, 0)),
            pl.BlockSpec((1, 1), lambda n: (0, 0)),
        ),
        out_shape=(
            jax.ShapeDtypeStruct((_NIMG, 1, _HW), jnp.int32),
            jax.ShapeDtypeStruct((_NIMG, _D, _HW), jnp.float32),
            jax.ShapeDtypeStruct((1, 1), jnp.float32),
        ),
        scratch_shapes=[pltpu.VMEM((_NCODES, 1), jnp.float32)],
    )(z_r, embedding, embm2, embT)
    z_q = zq.reshape(_NIMG, _D, 32, 32)
    indices = idx3.reshape(_NIMG, 32, 32)
    return (z_q, loss[0, 0], indices)


# speedup vs baseline: 1.6915x; 1.6915x over previous
import jax
import jax.numpy as jnp
from jax import lax
from jax.experimental import pallas as pl
from jax.experimental.pallas import tpu as pltpu

_NCODES = 1024
_HW = 1024
_D = 64
_NIMG = 16
_NELEM = float(_NIMG * _HW * _D)


def _vq_body(z_ref, en_ref, embm2_ref, embT_ref, idx_ref, zq_ref, loss_ref):
    n = pl.program_id(0)
    z2d = z_ref[0]
    zn = jnp.sum(z2d * z2d, axis=0, keepdims=True)
    en = en_ref[...]
    dotm2 = lax.dot_general(
        embm2_ref[...], z2d, (((1,), (0,)), ((), ())),
        preferred_element_type=jnp.float32,
    )
    dist = (zn + en) + dotm2
    m = jnp.min(dist, axis=0, keepdims=True)
    kio = lax.broadcasted_iota(jnp.int32, (_NCODES, _HW), 0)
    idx = jnp.min(jnp.where(dist == m, kio, _NCODES), axis=0, keepdims=True)
    idx_ref[0] = idx
    oh = jnp.where(kio == idx, 1.0, 0.0)
    zqT = lax.dot_general(
        embT_ref[...], oh, (((1,), (0,)), ((), ())),
        preferred_element_type=jnp.float32,
    )
    zq_ref[0] = zqT
    part = jnp.sum(m, keepdims=True).reshape(1, 1)
    prev = jnp.where(n == 0, jnp.zeros((1, 1), jnp.float32), loss_ref[...])
    total = prev + part
    loss_ref[...] = jnp.where(n == _NIMG - 1, total / _NELEM, total)


def kernel(z_e, embedding):
    z_r = z_e.reshape(_NIMG, _D, _HW)
    en_in = jnp.sum(embedding**2, axis=1, keepdims=True)
    embm2 = embedding * -2.0
    embT = embedding.T
    idx3, zq, loss = pl.pallas_call(
        _vq_body,
        grid=(_NIMG,),
        in_specs=[
            pl.BlockSpec((1, _D, _HW), lambda n: (n, 0, 0)),
            pl.BlockSpec((_NCODES, 1), lambda n: (0, 0)),
            pl.BlockSpec((_NCODES, _D), lambda n: (0, 0)),
            pl.BlockSpec((_D, _NCODES), lambda n: (0, 0)),
        ],
        out_specs=(
            pl.BlockSpec((1, 1, _HW), lambda n: (n, 0, 0)),
            pl.BlockSpec((1, _D, _HW), lambda n: (n, 0, 0)),
            pl.BlockSpec((1, 1), lambda n: (0, 0)),
        ),
        out_shape=(
            jax.ShapeDtypeStruct((_NIMG, 1, _HW), jnp.int32),
            jax.ShapeDtypeStruct((_NIMG, _D, _HW), jnp.float32),
            jax.ShapeDtypeStruct((1, 1), jnp.float32),
        ),
    )(z_r, en_in, embm2, embT)
    z_q = zq.reshape(_NIMG, _D, 32, 32)
    indices = idx3.reshape(_NIMG, 32, 32)
    return (z_q, loss[0, 0], indices)
